# Initial kernel scaffold; baseline (speedup 1.0000x reference)
import jax
import jax.numpy as jnp
from jax.experimental import pallas as pl


def kernel(f_atoms, f_bonds, a2b, b2a, b2revb, W_i, W_h0, W_h1, W_o, b_o, W_il, b_il, W_jl, b_jl):
    raise NotImplementedError("write your pallas kernel here")


# speedup vs baseline: 1.6314x; 1.6314x over previous
import functools

import jax
import jax.numpy as jnp
from jax import lax
from jax.experimental import pallas as pl
from jax.experimental.pallas import tpu as pltpu
from jax.experimental.pallas import tpu_sc as plsc

N_ATOMS = 10000
N_EDGES = 320000
MAX_NB = 32
ATOM_FDIM = 128
BOND_FDIM = 144
HIDDEN = 64
HP = 128
N_MOLS = 100
APM = 100

_NC = 2
_NS = 16
_NW = _NC * _NS


_SUB = 50
_C = 400


def _make_gather(K):
    per_w = K // _NW
    n_chunks = per_w // _C
    n_sub = _C // _SUB
    idx_rows = per_w // _SUB
    mesh = plsc.VectorSubcoreMesh(core_axis_name="c", subcore_axis_name="s")

    @functools.partial(
        pl.kernel,
        mesh=mesh,
        out_type=jax.ShapeDtypeStruct((K, HP), jnp.float32),
        scratch_types=[
            pltpu.VMEM((idx_rows, _SUB), jnp.int32),
            pltpu.VMEM((_C, HP), jnp.float32),
            pltpu.SemaphoreType.DMA,
        ],
    )
    def gather(table_hbm, idx_hbm, out_hbm, idx_v, rows_v, sem):
        wid = lax.axis_index("s") * _NC + lax.axis_index("c")
        base = pl.multiple_of(wid * per_w, 8 * _SUB)
        pltpu.sync_copy(
            idx_hbm.at[pl.ds(pl.multiple_of(wid * idx_rows, 8), idx_rows)],
            idx_v)

        def chunk(c, carry):
            off = pl.multiple_of(base + c * _C, 8)
            copies = []
            for j in range(n_sub):
                copies.append(pltpu.async_copy(
                    table_hbm.at[idx_v.at[c * n_sub + j]],
                    rows_v.at[pl.ds(j * _SUB, _SUB)], sem))
            for cp in copies:
                cp.wait()
            pltpu.sync_copy(rows_v, out_hbm.at[pl.ds(off, _C)])
            return carry

        lax.fori_loop(0, n_chunks, chunk, 0)

    return gather


_gather_edges = _make_gather(K=N_EDGES)


def _mm_body(x_ref, w_ref, o_ref):
    o_ref[...] = jnp.dot(x_ref[...], w_ref[...],
                         preferred_element_type=jnp.float32)


def _input_matmul(f_bonds, W_ip):
    BE = 4000
    return pl.pallas_call(
        _mm_body,
        grid=(N_EDGES // BE,),
        in_specs=[pl.BlockSpec((BE, BOND_FDIM), lambda i: (i, 0)),
                  pl.BlockSpec((BOND_FDIM, HP), lambda i: (0, 0))],
        out_specs=pl.BlockSpec((BE, HP), lambda i: (i, 0)),
        out_shape=jax.ShapeDtypeStruct((N_EDGES, HP), jnp.float32),
    )(f_bonds, W_ip)


def _segsum_body(x_ref, o_ref):
    o_ref[...] = jnp.sum(jnp.maximum(x_ref[...], 0.0), axis=1)


def _segsum_relu(nm3):
    BA = 400
    return pl.pallas_call(
        _segsum_body,
        grid=(N_ATOMS // BA,),
        in_specs=[pl.BlockSpec((BA, MAX_NB, HP), lambda i: (i, 0, 0))],
        out_specs=pl.BlockSpec((BA, HP), lambda i: (i, 0)),
        out_shape=jax.ShapeDtypeStruct((N_ATOMS, HP), jnp.float32),
    )(nm3)


def _update_body(z0_ref, ga_ref, gr_ref, w_ref, o_ref):
    m = ga_ref[...] - jnp.maximum(gr_ref[...], 0.0)
    o_ref[...] = z0_ref[...] + jnp.dot(m, w_ref[...],
                                       preferred_element_type=jnp.float32)


def _update(z0, ga, gr, W_hp):
    BE = 4000
    return pl.pallas_call(
        _update_body,
        grid=(N_EDGES // BE,),
        in_specs=[pl.BlockSpec((BE, HP), lambda i: (i, 0)),
                  pl.BlockSpec((BE, HP), lambda i: (i, 0)),
                  pl.BlockSpec((BE, HP), lambda i: (i, 0)),
                  pl.BlockSpec((HP, HP), lambda i: (0, 0))],
        out_specs=pl.BlockSpec((BE, HP), lambda i: (i, 0)),
        out_shape=jax.ShapeDtypeStruct((N_EDGES, HP), jnp.float32),
    )(z0, ga, gr, W_hp)


def _final_body(fa_ref, am_ref, woa_ref, wom_ref, bo_ref, wil_ref, bil_ref,
                wjl_ref, bjl_ref, o_ref):
    ah = jnp.dot(fa_ref[...], woa_ref[...],
                 preferred_element_type=jnp.float32)
    ah = ah + jnp.dot(am_ref[...], wom_ref[...],
                      preferred_element_type=jnp.float32)
    ah = jnp.maximum(ah + bo_ref[...], 0.0)
    s = jax.nn.sigmoid(jnp.dot(ah, wjl_ref[...],
                               preferred_element_type=jnp.float32)
                       + bjl_ref[...])
    t = jnp.dot(ah, wil_ref[...],
                preferred_element_type=jnp.float32) + bil_ref[...]
    u = s * t
    m_ids = lax.broadcasted_iota(jnp.int32, (N_MOLS, N_ATOMS), 0)
    a_ids = lax.broadcasted_iota(jnp.int32, (N_MOLS, N_ATOMS), 1)
    d = a_ids - APM * m_ids
    sel = jnp.where((d >= 0) & (d < APM), 1.0, 0.0)
    o_ref[...] = jnp.dot(sel, u, preferred_element_type=jnp.float32)


def _final(f_atoms, a_msg, W_oa, W_omp, b_o, W_il, b_il, W_jl, b_jl):
    def full(s):
        return pl.BlockSpec(s, lambda: tuple(0 for _ in s))
    return pl.pallas_call(
        _final_body,
        in_specs=[full((N_ATOMS, ATOM_FDIM)), full((N_ATOMS, HP)),
                  full((ATOM_FDIM, HIDDEN)), full((HP, HIDDEN)),
                  full((1, HIDDEN)),
                  full((HIDDEN, HIDDEN)), full((1, HIDDEN)),
                  full((HIDDEN, HIDDEN)), full((1, HIDDEN))],
        out_specs=full((N_MOLS, HIDDEN)),
        out_shape=jax.ShapeDtypeStruct((N_MOLS, HIDDEN), jnp.float32),
    )(f_atoms, a_msg, W_oa, W_omp, b_o.reshape(1, -1), W_il,
      b_il.reshape(1, -1), W_jl, b_jl.reshape(1, -1))


def _pad_cols(w, n):
    return jnp.concatenate(
        [w, jnp.zeros((w.shape[0], n - w.shape[1]), w.dtype)], axis=1)


def kernel(f_atoms, f_bonds, a2b, b2a, b2revb, W_i, W_h0, W_h1, W_o, b_o,
           W_il, b_il, W_jl, b_jl):
    idx_a2b = a2b.astype(jnp.int32).reshape(N_EDGES // _SUB, _SUB)
    idx_rev = b2revb.astype(jnp.int32).reshape(N_EDGES // _SUB, _SUB)
    idx_b2a = b2a.astype(jnp.int32).reshape(N_EDGES // _SUB, _SUB)

    W_ip = _pad_cols(W_i, HP)
    W_h0p = _pad_cols(jnp.pad(W_h0, ((0, HP - HIDDEN), (0, 0))), HP)
    W_h1p = _pad_cols(jnp.pad(W_h1, ((0, HP - HIDDEN), (0, 0))), HP)
    W_oa = W_o[:ATOM_FDIM]
    W_omp = jnp.pad(W_o[ATOM_FDIM:], ((0, HP - HIDDEN), (0, 0)))

    z0 = _input_matmul(f_bonds, W_ip)
    z = z0
    for W_hp in (W_h0p, W_h1p):
        nm = _gather_edges(z, idx_a2b)
        a_msg = _segsum_relu(nm.reshape(N_ATOMS, MAX_NB, HP))
        gr = _gather_edges(z, idx_rev)
        ga = _gather_edges(a_msg, idx_b2a)
        z = _update(z0, ga, gr, W_hp)
    nm = _gather_edges(z, idx_a2b)
    a_msg = _segsum_relu(nm.reshape(N_ATOMS, MAX_NB, HP))
    return _final(f_atoms, a_msg, W_oa, W_omp, b_o, W_il, b_il, W_jl, b_jl)


# speedup vs baseline: 1.9653x; 1.2047x over previous
import functools

import jax
import jax.numpy as jnp
from jax import lax
from jax.experimental import pallas as pl
from jax.experimental.pallas import tpu as pltpu
from jax.experimental.pallas import tpu_sc as plsc

N_ATOMS = 10000
N_EDGES = 320000
MAX_NB = 32
ATOM_FDIM = 128
BOND_FDIM = 144
HIDDEN = 64
HP = 128
N_MOLS = 100
APM = 100

_NC = 2
_NS = 16
_NW = _NC * _NS


_SUB = 50
_C = 400


def _make_gather(K):
    per_w = K // _NW
    n_chunks = per_w // _C
    n_sub = _C // _SUB
    idx_rows = per_w // _SUB
    mesh = plsc.VectorSubcoreMesh(core_axis_name="c", subcore_axis_name="s")

    @functools.partial(
        pl.kernel,
        mesh=mesh,
        out_type=jax.ShapeDtypeStruct((K, HP), jnp.float32),
        scratch_types=[
            pltpu.VMEM((idx_rows, _SUB), jnp.int32),
            pltpu.VMEM((_C, HP), jnp.float32),
            pltpu.SemaphoreType.DMA,
        ],
    )
    def gather(table_hbm, idx_hbm, out_hbm, idx_v, rows_v, sem):
        wid = lax.axis_index("s") * _NC + lax.axis_index("c")
        base = pl.multiple_of(wid * per_w, 8 * _SUB)
        pltpu.sync_copy(
            idx_hbm.at[pl.ds(pl.multiple_of(wid * idx_rows, 8), idx_rows)],
            idx_v)

        def chunk(c, carry):
            off = pl.multiple_of(base + c * _C, 8)
            copies = []
            for j in range(n_sub):
                copies.append(pltpu.async_copy(
                    table_hbm.at[idx_v.at[c * n_sub + j]],
                    rows_v.at[pl.ds(j * _SUB, _SUB)], sem))
            for cp in copies:
                cp.wait()
            pltpu.sync_copy(rows_v, out_hbm.at[pl.ds(off, _C)])
            return carry

        lax.fori_loop(0, n_chunks, chunk, 0)

    return gather


_gather_edges = _make_gather(K=N_EDGES)


_ATPC = 8
_AT_BASE = 312
_AT_LAST = 328


def _make_segsum():
    mesh = plsc.VectorSubcoreMesh(core_axis_name="c", subcore_axis_name="s")

    @functools.partial(
        pl.kernel,
        mesh=mesh,
        out_type=jax.ShapeDtypeStruct((N_ATOMS, HP), jnp.float32),
        scratch_types=[
            pltpu.VMEM((_AT_LAST, MAX_NB), jnp.int32),
            pltpu.VMEM((_ATPC * MAX_NB, HP), jnp.float32),
            pltpu.VMEM((_ATPC, HP), jnp.float32),
            pltpu.SemaphoreType.DMA,
        ],
    )
    def segsum(z_hbm, a2b_hbm, out_hbm, idx_v, buf_v, ob_v, sem):
        wid = lax.axis_index("s") * _NC + lax.axis_index("c")
        base = pl.multiple_of(wid * _AT_BASE, 8)
        pltpu.sync_copy(a2b_hbm.at[pl.ds(base, _AT_LAST)], idx_v)
        zeros = jnp.zeros((16,), jnp.float32)
        for j in range(_ATPC):
            for v in range(4, 8):
                ob_v[j, pl.ds(v * 16, 16)] = zeros
        n_chunks = jnp.where(wid == _NW - 1, _AT_LAST // _ATPC,
                             _AT_BASE // _ATPC)

        def chunk(c, carry):
            copies = []
            for j in range(_ATPC):
                copies.append(pltpu.async_copy(
                    z_hbm.at[idx_v.at[c * _ATPC + j]],
                    buf_v.at[pl.ds(j * MAX_NB, MAX_NB)], sem))
            for cp in copies:
                cp.wait()
            for j in range(_ATPC):
                def rbody(it, accs, j=j):
                    new = list(accs)
                    for rr in range(4):
                        row = j * MAX_NB + it * 4 + rr
                        for v in range(4):
                            x = buf_v[row, pl.ds(v * 16, 16)]
                            new[v] = new[v] + jnp.maximum(x, 0.0)
                    return tuple(new)
                accs = lax.fori_loop(0, MAX_NB // 4, rbody, (zeros,) * 4)
                for v in range(4):
                    ob_v[j, pl.ds(v * 16, 16)] = accs[v]
            pltpu.sync_copy(
                ob_v,
                out_hbm.at[pl.ds(pl.multiple_of(base + c * _ATPC, 8),
                                 _ATPC)])
            return carry

        lax.fori_loop(0, n_chunks, chunk, 0)

    return segsum


_segsum_sc = _make_segsum()


_C2 = 200
_SUB2 = 50


def _make_combine():
    per_w = N_EDGES // _NW
    idx_rows = per_w // _SUB2
    n_chunks = per_w // _C2
    n_sub = _C2 // _SUB2
    mesh = plsc.VectorSubcoreMesh(core_axis_name="c", subcore_axis_name="s")

    @functools.partial(
        pl.kernel,
        mesh=mesh,
        out_type=jax.ShapeDtypeStruct((N_EDGES, HP), jnp.float32),
        scratch_types=[
            pltpu.VMEM((idx_rows, _SUB2), jnp.int32),
            pltpu.VMEM((idx_rows, _SUB2), jnp.int32),
            pltpu.VMEM((_C2, HP), jnp.float32),
            pltpu.VMEM((_C2, HP), jnp.float32),
            pltpu.SemaphoreType.DMA,
        ],
    )
    def combine(z_hbm, amsg_hbm, idxr_hbm, idxa_hbm, out_hbm,
                idxr_v, idxa_v, gr_v, ga_v, sem):
        wid = lax.axis_index("s") * _NC + lax.axis_index("c")
        base = pl.multiple_of(wid * per_w, 8)
        irow = pl.multiple_of(wid * idx_rows, 8)
        pltpu.sync_copy(idxr_hbm.at[pl.ds(irow, idx_rows)], idxr_v)
        pltpu.sync_copy(idxa_hbm.at[pl.ds(irow, idx_rows)], idxa_v)

        def chunk(c, carry):
            copies = []
            for j in range(n_sub):
                copies.append(pltpu.async_copy(
                    z_hbm.at[idxr_v.at[c * n_sub + j]],
                    gr_v.at[pl.ds(j * _SUB2, _SUB2)], sem))
                copies.append(pltpu.async_copy(
                    amsg_hbm.at[idxa_v.at[c * n_sub + j]],
                    ga_v.at[pl.ds(j * _SUB2, _SUB2)], sem))
            for cp in copies:
                cp.wait()

            def rbody(it, carry):
                for rr in range(2):
                    row = it * 2 + rr
                    for v in range(4):
                        g = ga_v[row, pl.ds(v * 16, 16)]
                        r_ = gr_v[row, pl.ds(v * 16, 16)]
                        ga_v[row, pl.ds(v * 16, 16)] = (
                            g - jnp.maximum(r_, 0.0))
                return carry

            lax.fori_loop(0, _C2 // 2, rbody, 0)
            pltpu.sync_copy(
                ga_v,
                out_hbm.at[pl.ds(pl.multiple_of(base + c * _C2, 8), _C2)])
            return carry

        lax.fori_loop(0, n_chunks, chunk, 0)

    return combine


_combine_sc = _make_combine()


def _mm_body(x_ref, w_ref, o_ref):
    o_ref[...] = jnp.dot(x_ref[...], w_ref[...],
                         preferred_element_type=jnp.float32)


def _input_matmul(f_bonds, W_ip):
    BE = 4000
    return pl.pallas_call(
        _mm_body,
        grid=(N_EDGES // BE,),
        in_specs=[pl.BlockSpec((BE, BOND_FDIM), lambda i: (i, 0)),
                  pl.BlockSpec((BOND_FDIM, HP), lambda i: (0, 0))],
        out_specs=pl.BlockSpec((BE, HP), lambda i: (i, 0)),
        out_shape=jax.ShapeDtypeStruct((N_EDGES, HP), jnp.float32),
    )(f_bonds, W_ip)


def _segsum_body(x_ref, o_ref):
    o_ref[...] = jnp.sum(jnp.maximum(x_ref[...], 0.0), axis=1)


def _segsum_relu(nm3):
    BA = 400
    return pl.pallas_call(
        _segsum_body,
        grid=(N_ATOMS // BA,),
        in_specs=[pl.BlockSpec((BA, MAX_NB, HP), lambda i: (i, 0, 0))],
        out_specs=pl.BlockSpec((BA, HP), lambda i: (i, 0)),
        out_shape=jax.ShapeDtypeStruct((N_ATOMS, HP), jnp.float32),
    )(nm3)


def _update_body(z0_ref, m_ref, w_ref, o_ref):
    o_ref[...] = z0_ref[...] + jnp.dot(m_ref[...], w_ref[...],
                                       preferred_element_type=jnp.float32)


def _update(z0, m, W_hp):
    BE = 4000
    return pl.pallas_call(
        _update_body,
        grid=(N_EDGES // BE,),
        in_specs=[pl.BlockSpec((BE, HP), lambda i: (i, 0)),
                  pl.BlockSpec((BE, HP), lambda i: (i, 0)),
                  pl.BlockSpec((HP, HP), lambda i: (0, 0))],
        out_specs=pl.BlockSpec((BE, HP), lambda i: (i, 0)),
        out_shape=jax.ShapeDtypeStruct((N_EDGES, HP), jnp.float32),
    )(z0, m, W_hp)


def _final_body(fa_ref, am_ref, woa_ref, wom_ref, bo_ref, wil_ref, bil_ref,
                wjl_ref, bjl_ref, o_ref):
    ah = jnp.dot(fa_ref[...], woa_ref[...],
                 preferred_element_type=jnp.float32)
    ah = ah + jnp.dot(am_ref[...], wom_ref[...],
                      preferred_element_type=jnp.float32)
    ah = jnp.maximum(ah + bo_ref[...], 0.0)
    s = jax.nn.sigmoid(jnp.dot(ah, wjl_ref[...],
                               preferred_element_type=jnp.float32)
                       + bjl_ref[...])
    t = jnp.dot(ah, wil_ref[...],
                preferred_element_type=jnp.float32) + bil_ref[...]
    u = s * t
    m_ids = lax.broadcasted_iota(jnp.int32, (N_MOLS, N_ATOMS), 0)
    a_ids = lax.broadcasted_iota(jnp.int32, (N_MOLS, N_ATOMS), 1)
    d = a_ids - APM * m_ids
    sel = jnp.where((d >= 0) & (d < APM), 1.0, 0.0)
    o_ref[...] = jnp.dot(sel, u, preferred_element_type=jnp.float32)


def _final(f_atoms, a_msg, W_oa, W_omp, b_o, W_il, b_il, W_jl, b_jl):
    def full(s):
        return pl.BlockSpec(s, lambda: tuple(0 for _ in s))
    return pl.pallas_call(
        _final_body,
        in_specs=[full((N_ATOMS, ATOM_FDIM)), full((N_ATOMS, HP)),
                  full((ATOM_FDIM, HIDDEN)), full((HP, HIDDEN)),
                  full((1, HIDDEN)),
                  full((HIDDEN, HIDDEN)), full((1, HIDDEN)),
                  full((HIDDEN, HIDDEN)), full((1, HIDDEN))],
        out_specs=full((N_MOLS, HIDDEN)),
        out_shape=jax.ShapeDtypeStruct((N_MOLS, HIDDEN), jnp.float32),
    )(f_atoms, a_msg, W_oa, W_omp, b_o.reshape(1, -1), W_il,
      b_il.reshape(1, -1), W_jl, b_jl.reshape(1, -1))


def _pad_cols(w, n):
    return jnp.concatenate(
        [w, jnp.zeros((w.shape[0], n - w.shape[1]), w.dtype)], axis=1)


def kernel(f_atoms, f_bonds, a2b, b2a, b2revb, W_i, W_h0, W_h1, W_o, b_o,
           W_il, b_il, W_jl, b_jl):
    a2b_i = a2b.astype(jnp.int32)
    idx_rev = b2revb.astype(jnp.int32).reshape(N_EDGES // _SUB2, _SUB2)
    idx_b2a = b2a.astype(jnp.int32).reshape(N_EDGES // _SUB2, _SUB2)

    W_ip = _pad_cols(W_i, HP)
    W_h0p = _pad_cols(jnp.pad(W_h0, ((0, HP - HIDDEN), (0, 0))), HP)
    W_h1p = _pad_cols(jnp.pad(W_h1, ((0, HP - HIDDEN), (0, 0))), HP)
    W_oa = W_o[:ATOM_FDIM]
    W_omp = jnp.pad(W_o[ATOM_FDIM:], ((0, HP - HIDDEN), (0, 0)))

    z0 = _input_matmul(f_bonds, W_ip)
    z = z0
    for W_hp in (W_h0p, W_h1p):
        a_msg = _segsum_sc(z, a2b_i)
        m = _combine_sc(z, a_msg, idx_rev, idx_b2a)
        z = _update(z0, m, W_hp)
    a_msg = _segsum_sc(z, a2b_i)
    return _final(f_atoms, a_msg, W_oa, W_omp, b_o, W_il, b_il, W_jl, b_jl)


# speedup vs baseline: 2.1021x; 1.0696x over previous
import functools

import jax
import jax.numpy as jnp
from jax import lax
from jax.experimental import pallas as pl
from jax.experimental.pallas import tpu as pltpu
from jax.experimental.pallas import tpu_sc as plsc

N_ATOMS = 10000
N_EDGES = 320000
MAX_NB = 32
ATOM_FDIM = 128
BOND_FDIM = 144
HIDDEN = 64
HP = 128
N_MOLS = 100
APM = 100

_NC = 2
_NS = 16
_NW = _NC * _NS


_ATPC = 8
_AT_BASE = 312
_AT_LAST = 328


def _make_segsum():
    mesh = plsc.VectorSubcoreMesh(core_axis_name="c", subcore_axis_name="s")

    @functools.partial(
        pl.kernel,
        mesh=mesh,
        out_type=jax.ShapeDtypeStruct((N_ATOMS, HP), jnp.float32),
        scratch_types=[
            pltpu.VMEM((_AT_LAST, MAX_NB), jnp.int32),
            pltpu.VMEM((_ATPC * MAX_NB, HP), jnp.float32),
            pltpu.VMEM((_ATPC * MAX_NB, HP), jnp.float32),
            pltpu.VMEM((_ATPC, HP), jnp.float32),
            pltpu.SemaphoreType.DMA,
            pltpu.SemaphoreType.DMA,
        ],
    )
    def segsum(z_hbm, a2b_hbm, out_hbm, idx_v, buf0_v, buf1_v, ob_v,
               sem0, sem1):
        wid = lax.axis_index("s") * _NC + lax.axis_index("c")
        base = pl.multiple_of(wid * _AT_BASE, 8)
        pltpu.sync_copy(a2b_hbm.at[pl.ds(base, _AT_LAST)], idx_v)
        zeros = jnp.zeros((16,), jnp.float32)
        for j in range(_ATPC):
            for v in range(4, 8):
                ob_v[j, pl.ds(v * 16, 16)] = zeros
        n_chunks = jnp.where(wid == _NW - 1, _AT_LAST // _ATPC,
                             _AT_BASE // _ATPC)
        bufs = (buf0_v, buf1_v)
        sems = (sem0, sem1)

        def fire(c, b):
            buf = bufs[b]
            for j in range(_ATPC):
                pltpu.async_copy(
                    z_hbm.at[idx_v.at[c * _ATPC + j]],
                    buf.at[pl.ds(j * MAX_NB, MAX_NB)], sems[b])

        def drain(b):
            pltpu.make_async_copy(
                z_hbm.at[pl.ds(0, _ATPC * MAX_NB)], bufs[b],
                sems[b]).wait()

        @pl.when(n_chunks > 0)
        def _():
            fire(0, 0)

        def pair(t, carry):
            for b in range(2):
                c = t * 2 + b

                @pl.when(c + 1 < n_chunks)
                def _(c=c, b=b):
                    fire(c + 1, 1 - b)

                @pl.when(c < n_chunks)
                def _(c=c, b=b):
                    drain(b)
                    buf = bufs[b]
                    for j in range(_ATPC):
                        def rbody(it, accs, j=j, buf=buf):
                            new = list(accs)
                            for rr in range(4):
                                row = j * MAX_NB + it * 4 + rr
                                for v in range(4):
                                    x = buf[row, pl.ds(v * 16, 16)]
                                    new[v] = new[v] + jnp.maximum(x, 0.0)
                            return tuple(new)
                        accs = lax.fori_loop(0, MAX_NB // 4, rbody,
                                             (zeros,) * 4)
                        for v in range(4):
                            ob_v[j, pl.ds(v * 16, 16)] = accs[v]
                    pltpu.sync_copy(
                        ob_v,
                        out_hbm.at[pl.ds(
                            pl.multiple_of(base + c * _ATPC, 8), _ATPC)])
            return carry

        lax.fori_loop(0, (_AT_LAST // _ATPC + 1) // 2, pair, 0)

    return segsum


_segsum_sc = _make_segsum()


_C2 = 200
_SUB2 = 50


def _make_combine():
    per_w = N_EDGES // _NW
    idx_rows = per_w // _SUB2
    n_chunks = per_w // _C2
    n_sub = _C2 // _SUB2
    mesh = plsc.VectorSubcoreMesh(core_axis_name="c", subcore_axis_name="s")

    @functools.partial(
        pl.kernel,
        mesh=mesh,
        out_type=jax.ShapeDtypeStruct((N_EDGES, HP), jnp.float32),
        scratch_types=[
            pltpu.VMEM((idx_rows, _SUB2), jnp.int32),
            pltpu.VMEM((idx_rows, _SUB2), jnp.int32),
            pltpu.VMEM((_C2, HP), jnp.float32),
            pltpu.VMEM((_C2, HP), jnp.float32),
            pltpu.SemaphoreType.DMA,
        ],
    )
    def combine(z_hbm, amsg_hbm, idxr_hbm, idxa_hbm, out_hbm,
                idxr_v, idxa_v, gr_v, ga_v, sem):
        wid = lax.axis_index("s") * _NC + lax.axis_index("c")
        base = pl.multiple_of(wid * per_w, 8)
        irow = pl.multiple_of(wid * idx_rows, 8)
        pltpu.sync_copy(idxr_hbm.at[pl.ds(irow, idx_rows)], idxr_v)
        pltpu.sync_copy(idxa_hbm.at[pl.ds(irow, idx_rows)], idxa_v)

        def chunk(c, carry):
            copies = []
            for j in range(n_sub):
                copies.append(pltpu.async_copy(
                    z_hbm.at[idxr_v.at[c * n_sub + j]],
                    gr_v.at[pl.ds(j * _SUB2, _SUB2)], sem))
                copies.append(pltpu.async_copy(
                    amsg_hbm.at[idxa_v.at[c * n_sub + j]],
                    ga_v.at[pl.ds(j * _SUB2, _SUB2)], sem))
            for cp in copies:
                cp.wait()

            def rbody(it, carry):
                for rr in range(2):
                    row = it * 2 + rr
                    for v in range(4):
                        g = ga_v[row, pl.ds(v * 16, 16)]
                        r_ = gr_v[row, pl.ds(v * 16, 16)]
                        ga_v[row, pl.ds(v * 16, 16)] = (
                            g - jnp.maximum(r_, 0.0))
                return carry

            lax.fori_loop(0, _C2 // 2, rbody, 0)
            pltpu.sync_copy(
                ga_v,
                out_hbm.at[pl.ds(pl.multiple_of(base + c * _C2, 8), _C2)])
            return carry

        lax.fori_loop(0, n_chunks, chunk, 0)

    return combine


_combine_sc = _make_combine()


def _mm_body(x_ref, w_ref, o_ref):
    o_ref[...] = jnp.dot(x_ref[...], w_ref[...],
                         preferred_element_type=jnp.float32)


def _input_matmul(f_bonds, W_ip):
    BE = 4000
    return pl.pallas_call(
        _mm_body,
        grid=(N_EDGES // BE,),
        in_specs=[pl.BlockSpec((BE, BOND_FDIM), lambda i: (i, 0)),
                  pl.BlockSpec((BOND_FDIM, HP), lambda i: (0, 0))],
        out_specs=pl.BlockSpec((BE, HP), lambda i: (i, 0)),
        out_shape=jax.ShapeDtypeStruct((N_EDGES, HP), jnp.float32),
    )(f_bonds, W_ip)


def _update_body(z0_ref, m_ref, w_ref, o_ref):
    o_ref[...] = z0_ref[...] + jnp.dot(m_ref[...], w_ref[...],
                                       preferred_element_type=jnp.float32)


def _update(z0, m, W_hp):
    BE = 4000
    return pl.pallas_call(
        _update_body,
        grid=(N_EDGES // BE,),
        in_specs=[pl.BlockSpec((BE, HP), lambda i: (i, 0)),
                  pl.BlockSpec((BE, HP), lambda i: (i, 0)),
                  pl.BlockSpec((HP, HP), lambda i: (0, 0))],
        out_specs=pl.BlockSpec((BE, HP), lambda i: (i, 0)),
        out_shape=jax.ShapeDtypeStruct((N_EDGES, HP), jnp.float32),
    )(z0, m, W_hp)


def _final_body(fa_ref, am_ref, woa_ref, wom_ref, bo_ref, wil_ref, bil_ref,
                wjl_ref, bjl_ref, o_ref):
    ah = jnp.dot(fa_ref[...], woa_ref[...],
                 preferred_element_type=jnp.float32)
    ah = ah + jnp.dot(am_ref[...], wom_ref[...],
                      preferred_element_type=jnp.float32)
    ah = jnp.maximum(ah + bo_ref[...], 0.0)
    s = jax.nn.sigmoid(jnp.dot(ah, wjl_ref[...],
                               preferred_element_type=jnp.float32)
                       + bjl_ref[...])
    t = jnp.dot(ah, wil_ref[...],
                preferred_element_type=jnp.float32) + bil_ref[...]
    u = s * t
    m_ids = lax.broadcasted_iota(jnp.int32, (N_MOLS, N_ATOMS), 0)
    a_ids = lax.broadcasted_iota(jnp.int32, (N_MOLS, N_ATOMS), 1)
    d = a_ids - APM * m_ids
    sel = jnp.where((d >= 0) & (d < APM), 1.0, 0.0)
    o_ref[...] = jnp.dot(sel, u, preferred_element_type=jnp.float32)


def _final(f_atoms, a_msg, W_oa, W_omp, b_o, W_il, b_il, W_jl, b_jl):
    def full(s):
        return pl.BlockSpec(s, lambda: tuple(0 for _ in s))
    return pl.pallas_call(
        _final_body,
        in_specs=[full((N_ATOMS, ATOM_FDIM)), full((N_ATOMS, HP)),
                  full((ATOM_FDIM, HIDDEN)), full((HP, HIDDEN)),
                  full((1, HIDDEN)),
                  full((HIDDEN, HIDDEN)), full((1, HIDDEN)),
                  full((HIDDEN, HIDDEN)), full((1, HIDDEN))],
        out_specs=full((N_MOLS, HIDDEN)),
        out_shape=jax.ShapeDtypeStruct((N_MOLS, HIDDEN), jnp.float32),
    )(f_atoms, a_msg, W_oa, W_omp, b_o.reshape(1, -1), W_il,
      b_il.reshape(1, -1), W_jl, b_jl.reshape(1, -1))


def _pad_cols(w, n):
    return jnp.concatenate(
        [w, jnp.zeros((w.shape[0], n - w.shape[1]), w.dtype)], axis=1)


def kernel(f_atoms, f_bonds, a2b, b2a, b2revb, W_i, W_h0, W_h1, W_o, b_o,
           W_il, b_il, W_jl, b_jl):
    a2b_i = a2b.astype(jnp.int32)
    idx_rev = b2revb.astype(jnp.int32).reshape(N_EDGES // _SUB2, _SUB2)
    idx_b2a = b2a.astype(jnp.int32).reshape(N_EDGES // _SUB2, _SUB2)

    W_ip = _pad_cols(W_i, HP)
    W_h0p = _pad_cols(jnp.pad(W_h0, ((0, HP - HIDDEN), (0, 0))), HP)
    W_h1p = _pad_cols(jnp.pad(W_h1, ((0, HP - HIDDEN), (0, 0))), HP)
    W_oa = W_o[:ATOM_FDIM]
    W_omp = jnp.pad(W_o[ATOM_FDIM:], ((0, HP - HIDDEN), (0, 0)))

    z0 = _input_matmul(f_bonds, W_ip)
    z = z0
    for W_hp in (W_h0p, W_h1p):
        a_msg = _segsum_sc(z, a2b_i)
        m = _combine_sc(z, a_msg, idx_rev, idx_b2a)
        z = _update(z0, m, W_hp)
    a_msg = _segsum_sc(z, a2b_i)
    return _final(f_atoms, a_msg, W_oa, W_omp, b_o, W_il, b_il, W_jl, b_jl)


# speedup vs baseline: 2.1717x; 1.0331x over previous
import functools

import jax
import jax.numpy as jnp
from jax import lax
from jax.experimental import pallas as pl
from jax.experimental.pallas import tpu as pltpu
from jax.experimental.pallas import tpu_sc as plsc

N_ATOMS = 10000
N_EDGES = 320000
MAX_NB = 32
ATOM_FDIM = 128
BOND_FDIM = 144
HIDDEN = 64
HP = 128
N_MOLS = 100
APM = 100

_NC = 2
_NS = 16
_NW = _NC * _NS


_ATPC = 8
_AT_BASE = 312
_AT_LAST = 328


def _make_segsum():
    mesh = plsc.VectorSubcoreMesh(core_axis_name="c", subcore_axis_name="s")

    @functools.partial(
        pl.kernel,
        mesh=mesh,
        out_type=jax.ShapeDtypeStruct((N_ATOMS, HP), jnp.float32),
        scratch_types=[
            pltpu.VMEM((_AT_LAST, MAX_NB), jnp.int32),
            pltpu.VMEM((_ATPC * MAX_NB, HP), jnp.float32),
            pltpu.VMEM((_ATPC * MAX_NB, HP), jnp.float32),
            pltpu.VMEM((_ATPC, HP), jnp.float32),
            pltpu.SemaphoreType.DMA,
            pltpu.SemaphoreType.DMA,
        ],
    )
    def segsum(z_hbm, a2b_hbm, out_hbm, idx_v, buf0_v, buf1_v, ob_v,
               sem0, sem1):
        wid = lax.axis_index("s") * _NC + lax.axis_index("c")
        base = pl.multiple_of(wid * _AT_BASE, 8)
        pltpu.sync_copy(a2b_hbm.at[pl.ds(base, _AT_LAST)], idx_v)
        zeros = jnp.zeros((16,), jnp.float32)
        for j in range(_ATPC):
            for v in range(4, 8):
                ob_v[j, pl.ds(v * 16, 16)] = zeros
        n_chunks = jnp.where(wid == _NW - 1, _AT_LAST // _ATPC,
                             _AT_BASE // _ATPC)
        bufs = (buf0_v, buf1_v)
        sems = (sem0, sem1)

        def fire(c, b):
            buf = bufs[b]
            for j in range(_ATPC):
                pltpu.async_copy(
                    z_hbm.at[idx_v.at[c * _ATPC + j]],
                    buf.at[pl.ds(j * MAX_NB, MAX_NB)], sems[b])

        def drain(b):
            pltpu.make_async_copy(
                z_hbm.at[pl.ds(0, _ATPC * MAX_NB)], bufs[b],
                sems[b]).wait()

        @pl.when(n_chunks > 0)
        def _():
            fire(0, 0)

        def pair(t, carry):
            for b in range(2):
                c = t * 2 + b

                @pl.when(c + 1 < n_chunks)
                def _(c=c, b=b):
                    fire(c + 1, 1 - b)

                @pl.when(c < n_chunks)
                def _(c=c, b=b):
                    drain(b)
                    buf = bufs[b]
                    for j in range(_ATPC):
                        def rbody(it, accs, j=j, buf=buf):
                            new = list(accs)
                            for rr in range(4):
                                row = j * MAX_NB + it * 4 + rr
                                for v in range(4):
                                    x = buf[row, pl.ds(v * 16, 16)]
                                    new[v] = new[v] + jnp.maximum(x, 0.0)
                            return tuple(new)
                        accs = lax.fori_loop(0, MAX_NB // 4, rbody,
                                             (zeros,) * 4)
                        for v in range(4):
                            ob_v[j, pl.ds(v * 16, 16)] = accs[v]
                    pltpu.sync_copy(
                        ob_v,
                        out_hbm.at[pl.ds(
                            pl.multiple_of(base + c * _ATPC, 8), _ATPC)])
            return carry

        lax.fori_loop(0, (_AT_LAST // _ATPC + 1) // 2, pair, 0)

    return segsum


_segsum_sc = _make_segsum()


_C2 = 200
_SUB2 = 50


def _make_combine():
    per_w = N_EDGES // _NW
    idx_rows = per_w // _SUB2
    n_chunks = per_w // _C2
    n_sub = _C2 // _SUB2
    mesh = plsc.VectorSubcoreMesh(core_axis_name="c", subcore_axis_name="s")

    @functools.partial(
        pl.kernel,
        mesh=mesh,
        out_type=jax.ShapeDtypeStruct((N_EDGES, HP), jnp.float32),
        scratch_types=[
            pltpu.VMEM((idx_rows, _SUB2), jnp.int32),
            pltpu.VMEM((idx_rows, _SUB2), jnp.int32),
            pltpu.VMEM((_C2, HP), jnp.float32),
            pltpu.VMEM((_C2, HP), jnp.float32),
        ] + [pltpu.SemaphoreType.DMA] * (2 * 4),
    )
    def combine(z_hbm, amsg_hbm, idxr_hbm, idxa_hbm, out_hbm,
                idxr_v, idxa_v, gr_v, ga_v, *sems):
        semr = sems[:4]
        sema = sems[4:]
        wid = lax.axis_index("s") * _NC + lax.axis_index("c")
        base = pl.multiple_of(wid * per_w, 8)
        irow = pl.multiple_of(wid * idx_rows, 8)
        pltpu.sync_copy(idxr_hbm.at[pl.ds(irow, idx_rows)], idxr_v)
        pltpu.sync_copy(idxa_hbm.at[pl.ds(irow, idx_rows)], idxa_v)

        def chunk(c, carry):
            copies = []
            for j in range(n_sub):
                cr = pltpu.async_copy(
                    z_hbm.at[idxr_v.at[c * n_sub + j]],
                    gr_v.at[pl.ds(j * _SUB2, _SUB2)], semr[j])
                ca = pltpu.async_copy(
                    amsg_hbm.at[idxa_v.at[c * n_sub + j]],
                    ga_v.at[pl.ds(j * _SUB2, _SUB2)], sema[j])
                copies.append((cr, ca))
            for j in range(n_sub):
                copies[j][0].wait()
                copies[j][1].wait()

                def rbody(it, carry, j=j):
                    row = j * _SUB2 + it
                    for v in range(4):
                        g = ga_v[row, pl.ds(v * 16, 16)]
                        r_ = gr_v[row, pl.ds(v * 16, 16)]
                        ga_v[row, pl.ds(v * 16, 16)] = (
                            g - jnp.maximum(r_, 0.0))
                    return carry

                lax.fori_loop(0, _SUB2, rbody, 0)
            pltpu.sync_copy(
                ga_v,
                out_hbm.at[pl.ds(pl.multiple_of(base + c * _C2, 8), _C2)])
            return carry

        lax.fori_loop(0, n_chunks, chunk, 0)

    return combine


_combine_sc = _make_combine()


def _mm_body(x_ref, w_ref, o_ref):
    o_ref[...] = jnp.dot(x_ref[...], w_ref[...],
                         preferred_element_type=jnp.float32)


def _input_matmul(f_bonds, W_ip):
    BE = 4000
    return pl.pallas_call(
        _mm_body,
        grid=(N_EDGES // BE,),
        in_specs=[pl.BlockSpec((BE, BOND_FDIM), lambda i: (i, 0)),
                  pl.BlockSpec((BOND_FDIM, HP), lambda i: (0, 0))],
        out_specs=pl.BlockSpec((BE, HP), lambda i: (i, 0)),
        out_shape=jax.ShapeDtypeStruct((N_EDGES, HP), jnp.float32),
    )(f_bonds, W_ip)


def _update_body(z0_ref, m_ref, w_ref, o_ref):
    o_ref[...] = z0_ref[...] + jnp.dot(m_ref[...], w_ref[...],
                                       preferred_element_type=jnp.float32)


def _update(z0, m, W_hp):
    BE = 4000
    return pl.pallas_call(
        _update_body,
        grid=(N_EDGES // BE,),
        in_specs=[pl.BlockSpec((BE, HP), lambda i: (i, 0)),
                  pl.BlockSpec((BE, HP), lambda i: (i, 0)),
                  pl.BlockSpec((HP, HP), lambda i: (0, 0))],
        out_specs=pl.BlockSpec((BE, HP), lambda i: (i, 0)),
        out_shape=jax.ShapeDtypeStruct((N_EDGES, HP), jnp.float32),
    )(z0, m, W_hp)


def _final_body(fa_ref, am_ref, woa_ref, wom_ref, bo_ref, wil_ref, bil_ref,
                wjl_ref, bjl_ref, o_ref):
    ah = jnp.dot(fa_ref[...], woa_ref[...],
                 preferred_element_type=jnp.float32)
    ah = ah + jnp.dot(am_ref[...], wom_ref[...],
                      preferred_element_type=jnp.float32)
    ah = jnp.maximum(ah + bo_ref[...], 0.0)
    s = jax.nn.sigmoid(jnp.dot(ah, wjl_ref[...],
                               preferred_element_type=jnp.float32)
                       + bjl_ref[...])
    t = jnp.dot(ah, wil_ref[...],
                preferred_element_type=jnp.float32) + bil_ref[...]
    u = s * t
    m_ids = lax.broadcasted_iota(jnp.int32, (N_MOLS, N_ATOMS), 0)
    a_ids = lax.broadcasted_iota(jnp.int32, (N_MOLS, N_ATOMS), 1)
    d = a_ids - APM * m_ids
    sel = jnp.where((d >= 0) & (d < APM), 1.0, 0.0)
    o_ref[...] = jnp.dot(sel, u, preferred_element_type=jnp.float32)


def _final(f_atoms, a_msg, W_oa, W_omp, b_o, W_il, b_il, W_jl, b_jl):
    def full(s):
        return pl.BlockSpec(s, lambda: tuple(0 for _ in s))
    return pl.pallas_call(
        _final_body,
        in_specs=[full((N_ATOMS, ATOM_FDIM)), full((N_ATOMS, HP)),
                  full((ATOM_FDIM, HIDDEN)), full((HP, HIDDEN)),
                  full((1, HIDDEN)),
                  full((HIDDEN, HIDDEN)), full((1, HIDDEN)),
                  full((HIDDEN, HIDDEN)), full((1, HIDDEN))],
        out_specs=full((N_MOLS, HIDDEN)),
        out_shape=jax.ShapeDtypeStruct((N_MOLS, HIDDEN), jnp.float32),
    )(f_atoms, a_msg, W_oa, W_omp, b_o.reshape(1, -1), W_il,
      b_il.reshape(1, -1), W_jl, b_jl.reshape(1, -1))


def _pad_cols(w, n):
    return jnp.concatenate(
        [w, jnp.zeros((w.shape[0], n - w.shape[1]), w.dtype)], axis=1)


def kernel(f_atoms, f_bonds, a2b, b2a, b2revb, W_i, W_h0, W_h1, W_o, b_o,
           W_il, b_il, W_jl, b_jl):
    a2b_i = a2b.astype(jnp.int32)
    idx_rev = b2revb.astype(jnp.int32).reshape(N_EDGES // _SUB2, _SUB2)
    idx_b2a = b2a.astype(jnp.int32).reshape(N_EDGES // _SUB2, _SUB2)

    W_ip = _pad_cols(W_i, HP)
    W_h0p = _pad_cols(jnp.pad(W_h0, ((0, HP - HIDDEN), (0, 0))), HP)
    W_h1p = _pad_cols(jnp.pad(W_h1, ((0, HP - HIDDEN), (0, 0))), HP)
    W_oa = W_o[:ATOM_FDIM]
    W_omp = jnp.pad(W_o[ATOM_FDIM:], ((0, HP - HIDDEN), (0, 0)))

    z0 = _input_matmul(f_bonds, W_ip)
    z = z0
    for W_hp in (W_h0p, W_h1p):
        a_msg = _segsum_sc(z, a2b_i)
        m = _combine_sc(z, a_msg, idx_rev, idx_b2a)
        z = _update(z0, m, W_hp)
    a_msg = _segsum_sc(z, a2b_i)
    return _final(f_atoms, a_msg, W_oa, W_omp, b_o, W_il, b_il, W_jl, b_jl)


# speedup vs baseline: 2.2007x; 1.0133x over previous
import functools

import jax
import jax.numpy as jnp
from jax import lax
from jax.experimental import pallas as pl
from jax.experimental.pallas import tpu as pltpu
from jax.experimental.pallas import tpu_sc as plsc

N_ATOMS = 10000
N_EDGES = 320000
MAX_NB = 32
ATOM_FDIM = 128
BOND_FDIM = 144
HIDDEN = 64
HP = 128
N_MOLS = 100
APM = 100

_NC = 2
_NS = 16
_NW = _NC * _NS


_ATPC = 8
_AT_BASE = 312
_AT_LAST = 328


def _make_segsum():
    mesh = plsc.VectorSubcoreMesh(core_axis_name="c", subcore_axis_name="s")

    @functools.partial(
        pl.kernel,
        mesh=mesh,
        out_type=jax.ShapeDtypeStruct((N_ATOMS, HP), jnp.float32),
        scratch_types=[
            pltpu.VMEM((_AT_LAST, MAX_NB), jnp.int32),
            pltpu.VMEM((_ATPC * MAX_NB, HP), jnp.float32),
            pltpu.VMEM((_ATPC * MAX_NB, HP), jnp.float32),
            pltpu.VMEM((_ATPC, HP), jnp.float32),
            pltpu.SemaphoreType.DMA,
            pltpu.SemaphoreType.DMA,
        ],
    )
    def segsum(z_hbm, a2b_hbm, out_hbm, idx_v, buf0_v, buf1_v, ob_v,
               sem0, sem1):
        wid = lax.axis_index("s") * _NC + lax.axis_index("c")
        base = pl.multiple_of(wid * _AT_BASE, 8)
        pltpu.sync_copy(a2b_hbm.at[pl.ds(base, _AT_LAST)], idx_v)
        zeros = jnp.zeros((16,), jnp.float32)
        for j in range(_ATPC):
            for v in range(4, 8):
                ob_v[j, pl.ds(v * 16, 16)] = zeros
        n_chunks = jnp.where(wid == _NW - 1, _AT_LAST // _ATPC,
                             _AT_BASE // _ATPC)
        bufs = (buf0_v, buf1_v)
        sems = (sem0, sem1)

        def fire(c, b):
            buf = bufs[b]
            for j in range(_ATPC):
                pltpu.async_copy(
                    z_hbm.at[idx_v.at[c * _ATPC + j]],
                    buf.at[pl.ds(j * MAX_NB, MAX_NB)], sems[b])

        def drain(b):
            pltpu.make_async_copy(
                z_hbm.at[pl.ds(0, _ATPC * MAX_NB)], bufs[b],
                sems[b]).wait()

        @pl.when(n_chunks > 0)
        def _():
            fire(0, 0)

        def pair(t, carry):
            for b in range(2):
                c = t * 2 + b

                @pl.when(c + 1 < n_chunks)
                def _(c=c, b=b):
                    fire(c + 1, 1 - b)

                @pl.when(c < n_chunks)
                def _(c=c, b=b):
                    drain(b)
                    buf = bufs[b]
                    for j in range(_ATPC):
                        def rbody(it, accs, j=j, buf=buf):
                            new = list(accs)
                            for rr in range(4):
                                row = j * MAX_NB + it * 4 + rr
                                for v in range(4):
                                    x = buf[row, pl.ds(v * 16, 16)]
                                    new[v] = new[v] + jnp.maximum(x, 0.0)
                            return tuple(new)
                        accs = lax.fori_loop(0, MAX_NB // 4, rbody,
                                             (zeros,) * 4)
                        for v in range(4):
                            ob_v[j, pl.ds(v * 16, 16)] = accs[v]
                    pltpu.sync_copy(
                        ob_v,
                        out_hbm.at[pl.ds(
                            pl.multiple_of(base + c * _ATPC, 8), _ATPC)])
            return carry

        lax.fori_loop(0, (_AT_LAST // _ATPC + 1) // 2, pair, 0)

    return segsum


_segsum_sc = _make_segsum()


_C2 = 200
_SUB2 = 50


def _make_combine():
    per_w = N_EDGES // _NW
    idx_rows = per_w // _SUB2
    n_chunks = per_w // _C2
    n_sub = _C2 // _SUB2
    mesh = plsc.VectorSubcoreMesh(core_axis_name="c", subcore_axis_name="s")

    @functools.partial(
        pl.kernel,
        mesh=mesh,
        out_type=jax.ShapeDtypeStruct((N_EDGES, HP), jnp.float32),
        scratch_types=[
            pltpu.VMEM((idx_rows, _SUB2), jnp.int32),
            pltpu.VMEM((idx_rows, _SUB2), jnp.int32),
            pltpu.VMEM((_C2, HP), jnp.float32),
            pltpu.VMEM((_C2, HP), jnp.float32),
        ] + [pltpu.SemaphoreType.DMA] * (2 * 4),
    )
    def combine(z_hbm, amsg_hbm, idxr_hbm, idxa_hbm, out_hbm,
                idxr_v, idxa_v, gr_v, ga_v, *sems):
        semr = sems[:4]
        sema = sems[4:]
        wid = lax.axis_index("s") * _NC + lax.axis_index("c")
        base = pl.multiple_of(wid * per_w, 8)
        irow = pl.multiple_of(wid * idx_rows, 8)
        pltpu.sync_copy(idxr_hbm.at[pl.ds(irow, idx_rows)], idxr_v)
        pltpu.sync_copy(idxa_hbm.at[pl.ds(irow, idx_rows)], idxa_v)

        def chunk(c, carry):
            copies = []
            for j in range(n_sub):
                cr = pltpu.async_copy(
                    z_hbm.at[idxr_v.at[c * n_sub + j]],
                    gr_v.at[pl.ds(j * _SUB2, _SUB2)], semr[j])
                ca = pltpu.async_copy(
                    amsg_hbm.at[idxa_v.at[c * n_sub + j]],
                    ga_v.at[pl.ds(j * _SUB2, _SUB2)], sema[j])
                copies.append((cr, ca))
            for j in range(n_sub):
                copies[j][0].wait()
                copies[j][1].wait()

                def rbody(it, carry, j=j):
                    row = j * _SUB2 + it
                    for v in range(4):
                        g = ga_v[row, pl.ds(v * 16, 16)]
                        r_ = gr_v[row, pl.ds(v * 16, 16)]
                        ga_v[row, pl.ds(v * 16, 16)] = (
                            g - jnp.maximum(r_, 0.0))
                    return carry

                lax.fori_loop(0, _SUB2, rbody, 0)
            pltpu.sync_copy(
                ga_v,
                out_hbm.at[pl.ds(pl.multiple_of(base + c * _C2, 8), _C2)])
            return carry

        lax.fori_loop(0, n_chunks, chunk, 0)

    return combine


_combine_sc = _make_combine()


def _mm_body(x_ref, w_ref, o_ref, ob_ref):
    y = jnp.dot(x_ref[...], w_ref[...], preferred_element_type=jnp.float32)
    o_ref[...] = y
    ob_ref[...] = y.astype(jnp.bfloat16)


def _input_matmul(f_bonds, W_ip):
    BE = 4000
    return pl.pallas_call(
        _mm_body,
        grid=(N_EDGES // BE,),
        in_specs=[pl.BlockSpec((BE, BOND_FDIM), lambda i: (i, 0)),
                  pl.BlockSpec((BOND_FDIM, HP), lambda i: (0, 0))],
        out_specs=[pl.BlockSpec((BE, HP), lambda i: (i, 0)),
                   pl.BlockSpec((BE, HP), lambda i: (i, 0))],
        out_shape=[jax.ShapeDtypeStruct((N_EDGES, HP), jnp.float32),
                   jax.ShapeDtypeStruct((N_EDGES, HP), jnp.bfloat16)],
    )(f_bonds, W_ip)


def _update_body(z0b_ref, m_ref, w_ref, o_ref):
    z0 = z0b_ref[...].astype(jnp.float32)
    o_ref[...] = z0 + jnp.dot(m_ref[...], w_ref[...],
                              preferred_element_type=jnp.float32)


def _update(z0b, m, W_hp):
    BE = 4000
    return pl.pallas_call(
        _update_body,
        grid=(N_EDGES // BE,),
        in_specs=[pl.BlockSpec((BE, HP), lambda i: (i, 0)),
                  pl.BlockSpec((BE, HP), lambda i: (i, 0)),
                  pl.BlockSpec((HP, HP), lambda i: (0, 0))],
        out_specs=pl.BlockSpec((BE, HP), lambda i: (i, 0)),
        out_shape=jax.ShapeDtypeStruct((N_EDGES, HP), jnp.float32),
    )(z0b, m, W_hp)


def _final_body(fa_ref, am_ref, woa_ref, wom_ref, bo_ref, wil_ref, bil_ref,
                wjl_ref, bjl_ref, o_ref):
    ah = jnp.dot(fa_ref[...], woa_ref[...],
                 preferred_element_type=jnp.float32)
    ah = ah + jnp.dot(am_ref[...], wom_ref[...],
                      preferred_element_type=jnp.float32)
    ah = jnp.maximum(ah + bo_ref[...], 0.0)
    s = jax.nn.sigmoid(jnp.dot(ah, wjl_ref[...],
                               preferred_element_type=jnp.float32)
                       + bjl_ref[...])
    t = jnp.dot(ah, wil_ref[...],
                preferred_element_type=jnp.float32) + bil_ref[...]
    u = s * t
    m_ids = lax.broadcasted_iota(jnp.int32, (N_MOLS, N_ATOMS), 0)
    a_ids = lax.broadcasted_iota(jnp.int32, (N_MOLS, N_ATOMS), 1)
    d = a_ids - APM * m_ids
    sel = jnp.where((d >= 0) & (d < APM), 1.0, 0.0)
    o_ref[...] = jnp.dot(sel, u, preferred_element_type=jnp.float32)


def _final(f_atoms, a_msg, W_oa, W_omp, b_o, W_il, b_il, W_jl, b_jl):
    def full(s):
        return pl.BlockSpec(s, lambda: tuple(0 for _ in s))
    return pl.pallas_call(
        _final_body,
        in_specs=[full((N_ATOMS, ATOM_FDIM)), full((N_ATOMS, HP)),
                  full((ATOM_FDIM, HIDDEN)), full((HP, HIDDEN)),
                  full((1, HIDDEN)),
                  full((HIDDEN, HIDDEN)), full((1, HIDDEN)),
                  full((HIDDEN, HIDDEN)), full((1, HIDDEN))],
        out_specs=full((N_MOLS, HIDDEN)),
        out_shape=jax.ShapeDtypeStruct((N_MOLS, HIDDEN), jnp.float32),
    )(f_atoms, a_msg, W_oa, W_omp, b_o.reshape(1, -1), W_il,
      b_il.reshape(1, -1), W_jl, b_jl.reshape(1, -1))


def _pad_cols(w, n):
    return jnp.concatenate(
        [w, jnp.zeros((w.shape[0], n - w.shape[1]), w.dtype)], axis=1)


def kernel(f_atoms, f_bonds, a2b, b2a, b2revb, W_i, W_h0, W_h1, W_o, b_o,
           W_il, b_il, W_jl, b_jl):
    a2b_i = a2b.astype(jnp.int32)
    idx_rev = b2revb.astype(jnp.int32).reshape(N_EDGES // _SUB2, _SUB2)
    idx_b2a = b2a.astype(jnp.int32).reshape(N_EDGES // _SUB2, _SUB2)

    W_ip = _pad_cols(W_i, HP)
    W_h0p = _pad_cols(jnp.pad(W_h0, ((0, HP - HIDDEN), (0, 0))), HP)
    W_h1p = _pad_cols(jnp.pad(W_h1, ((0, HP - HIDDEN), (0, 0))), HP)
    W_oa = W_o[:ATOM_FDIM]
    W_omp = jnp.pad(W_o[ATOM_FDIM:], ((0, HP - HIDDEN), (0, 0)))

    z0f, z0b = _input_matmul(f_bonds, W_ip)
    z = z0f
    for W_hp in (W_h0p, W_h1p):
        a_msg = _segsum_sc(z, a2b_i)
        m = _combine_sc(z, a_msg, idx_rev, idx_b2a)
        z = _update(z0b, m, W_hp)
    a_msg = _segsum_sc(z, a2b_i)
    return _final(f_atoms, a_msg, W_oa, W_omp, b_o, W_il, b_il, W_jl, b_jl)
